# Initial kernel scaffold; baseline (speedup 1.0000x reference)
import jax
import jax.numpy as jnp
from jax.experimental import pallas as pl


def kernel(X, W_self, W_neigh, b_sage, W1, b1, W2, b2, edge_index, conn_src, conn_dst):
    raise NotImplementedError("write your pallas kernel here")


# speedup vs baseline: 1.1029x; 1.1029x over previous
import functools

import jax
import jax.numpy as jnp
from jax.experimental import pallas as pl
from jax.experimental.pallas import tpu as pltpu

N_NODES = 10000
N_NET = 20000
N_EDGES = 320000
N_CONN = 80000
D = 128
H1 = 64

_RB1 = 1000
_RB2 = 1000


def _sage_dense_body(x_ref, aggp_ref, degp_ref, ws_ref, wn_ref, b_ref, o_ref):
    agg = aggp_ref[0] + aggp_ref[1]
    deg = degp_ref[0, :, 0] + degp_ref[1, :, 0]
    inv = 1.0 / jnp.clip(deg, 1.0, None)
    hn = agg * inv[:, None]
    h = (jnp.dot(x_ref[...], ws_ref[...], preferred_element_type=jnp.float32)
         + jnp.dot(hn, wn_ref[...], preferred_element_type=jnp.float32)
         + b_ref[...])
    o_ref[...] = jnp.where(h >= 0.0, h, 0.01 * h)


def _sage_dense(X, agg_parts, deg_parts, W_self, W_neigh, b_sage):
    grid = (N_NODES // _RB1,)
    return pl.pallas_call(
        _sage_dense_body,
        grid=grid,
        in_specs=[
            pl.BlockSpec((_RB1, D), lambda i: (i, 0)),
            pl.BlockSpec((2, _RB1, D), lambda i: (0, i, 0)),
            pl.BlockSpec((2, _RB1, 16), lambda i: (0, i, 0)),
            pl.BlockSpec((D, D), lambda i: (0, 0)),
            pl.BlockSpec((D, D), lambda i: (0, 0)),
            pl.BlockSpec((1, D), lambda i: (0, 0)),
        ],
        out_specs=pl.BlockSpec((_RB1, D), lambda i: (i, 0)),
        out_shape=jax.ShapeDtypeStruct((N_NODES, D), jnp.float32),
    )(X, agg_parts, deg_parts, W_self, W_neigh, b_sage.reshape(1, D))


def _mlp_body(xx_ref, w1_ref, b1_ref, w2_ref, b2_ref, o_ref):
    l1 = (jnp.dot(xx_ref[...], w1_ref[...], preferred_element_type=jnp.float32)
          + b1_ref[...])
    l1 = jnp.where(l1 >= 0.0, l1, 0.01 * l1)
    o_ref[...] = jnp.tanh(
        jnp.dot(l1, w2_ref[...], preferred_element_type=jnp.float32) + b2_ref[...])


def _mlp(xx, W1, b1, W2, b2):
    grid = (N_NET // _RB2,)
    return pl.pallas_call(
        _mlp_body,
        grid=grid,
        in_specs=[
            pl.BlockSpec((_RB2, D), lambda i: (i, 0)),
            pl.BlockSpec((D, H1), lambda i: (0, 0)),
            pl.BlockSpec((1, H1), lambda i: (0, 0)),
            pl.BlockSpec((H1, 1), lambda i: (0, 0)),
            pl.BlockSpec((1, 1), lambda i: (0, 0)),
        ],
        out_specs=pl.BlockSpec((_RB2, 1), lambda i: (i, 0)),
        out_shape=jax.ShapeDtypeStruct((N_NET, 1), jnp.float32),
    )(xx, W1, b1.reshape(1, H1), W2, b2.reshape(1, 1))


def kernel(X, W_self, W_neigh, b_sage, W1, b1, W2, b2, edge_index, conn_src, conn_dst):
    src = edge_index[0]
    dst = edge_index[1]
    msg = jnp.take(X, src, axis=0)
    agg = jax.ops.segment_sum(msg, dst, num_segments=N_NODES)
    deg = jax.ops.segment_sum(jnp.ones((N_EDGES,), jnp.float32), dst, num_segments=N_NODES)
    agg_parts = jnp.stack([agg, jnp.zeros_like(agg)])
    deg_parts = jnp.stack([jnp.broadcast_to(deg[:, None], (N_NODES, 16)),
                           jnp.zeros((N_NODES, 16), jnp.float32)])
    h = _sage_dense(X, agg_parts, deg_parts, W_self, W_neigh, b_sage)
    m = jnp.take(h, conn_src, axis=0)
    xx = jax.ops.segment_max(m, conn_dst, num_segments=N_NET)
    xx = jnp.where(jnp.isneginf(xx), 0.0, xx)
    return _mlp(xx, W1, b1, W2, b2)


# speedup vs baseline: 3.5534x; 3.2218x over previous
import functools

import jax
import jax.numpy as jnp
from jax import lax
from jax.experimental import pallas as pl
from jax.experimental.pallas import tpu as pltpu
from jax.experimental.pallas import tpu_sc as plsc

N_NODES = 10000
N_NET = 20000
N_EDGES = 320000
N_CONN = 80000
D = 128
H1 = 64

_NC, _NS = 2, 16
_NW = _NC * _NS
_EPW = N_EDGES // _NW
_CH = 80
_NCHUNK = _EPW // _CH
_RPT = 624
_TAIL0 = _NS * _RPT
_TAILN = N_NODES - _TAIL0


_HPAD = N_NODES + 16


def _seg_sum_body(x_hbm, src_hbm, dst_hbm, z128_hbm,
                  agg_out, deg_out0, deg_out1,
                  agg_sh, deg_st, src_v, dst_v, rows_v, hist_v, win_v, sem):
    cid = lax.axis_index("c")
    sid = lax.axis_index("s")
    wid = cid * _NS + sid
    pltpu.sync_copy(z128_hbm.at[pl.ds(0, _RPT)], agg_sh.at[pl.ds(sid * _RPT, _RPT)])

    @pl.when(sid == _NS - 1)
    def _zero_tail():
        pltpu.sync_copy(z128_hbm.at[pl.ds(0, _TAILN)], agg_sh.at[pl.ds(_TAIL0, _TAILN)])

    zeros16 = jnp.zeros((16,), jnp.float32)
    one0 = jnp.where(lax.iota(jnp.int32, 16) == 0, 1.0, 0.0).astype(jnp.float32)

    def zero_hist(i, c):
        hist_v[pl.ds(i * 16, 16)] = zeros16
        return c
    lax.fori_loop(0, _HPAD // 16, zero_hist, 0)
    plsc.subcore_barrier()

    base = wid * _EPW

    def step(i, c):
        off = base + i * _CH
        pltpu.sync_copy(src_hbm.at[pl.ds(off, _CH)], src_v)
        pltpu.sync_copy(dst_hbm.at[pl.ds(off, _CH)], dst_v)
        pltpu.async_copy(x_hbm.at[src_v], rows_v, sem).wait()
        pltpu.sync_copy(rows_v, agg_sh.at[dst_v], add=True)
        for g in range(_CH // 16):
            d_vec = dst_v[pl.ds(g * 16, 16)]
            for e in range(16):
                plsc.addupdate(hist_v.at[pl.ds(d_vec[e], 16)], one0)
        return c
    lax.fori_loop(0, _NCHUNK, step, 0)

    pltpu.sync_copy(hist_v.at[pl.ds(0, N_NODES)], deg_st.at[pl.ds(sid * N_NODES, N_NODES)])
    plsc.subcore_barrier()

    lo = sid * _RPT

    def reduce_window(lo, n, out_ref):
        pltpu.sync_copy(deg_st.at[pl.ds(lo, n)], win_v.at[pl.ds(0, n)])
        for j in range(1, _NS):
            pltpu.sync_copy(deg_st.at[pl.ds(j * N_NODES + lo, n)], win_v.at[pl.ds(n, n)])

            def acc(k, c):
                a = win_v[pl.ds(k * 16, 16)]
                b = win_v[pl.ds(n + k * 16, 16)]
                win_v[pl.ds(k * 16, 16)] = a + b
                return c
            lax.fori_loop(0, n // 16, acc, 0)
        pltpu.sync_copy(win_v.at[pl.ds(0, n)], out_ref.at[pl.ds(lo, n)])

    @pl.when(cid == 0)
    def _red0():
        reduce_window(lo, _RPT, deg_out0)

        @pl.when(sid == _NS - 1)
        def _tail0():
            reduce_window(_TAIL0, _TAILN, deg_out0)

    @pl.when(cid == 1)
    def _red1():
        reduce_window(lo, _RPT, deg_out1)

        @pl.when(sid == _NS - 1)
        def _tail1():
            reduce_window(_TAIL0, _TAILN, deg_out1)

    plsc.subcore_barrier()
    pltpu.sync_copy(agg_sh.at[pl.ds(sid * _RPT, _RPT)],
                    agg_out.at[cid, pl.ds(sid * _RPT, _RPT)])

    @pl.when(sid == _NS - 1)
    def _copy_tail():
        pltpu.sync_copy(agg_sh.at[pl.ds(_TAIL0, _TAILN)],
                        agg_out.at[cid, pl.ds(_TAIL0, _TAILN)])


def _seg_sum_sc(X, src, dst):
    mesh = plsc.VectorSubcoreMesh(core_axis_name="c", subcore_axis_name="s",
                                  num_cores=_NC, num_subcores=_NS)
    z128 = jnp.zeros((_RPT, D), jnp.float32)
    fn = pl.kernel(
        _seg_sum_body,
        out_type=[jax.ShapeDtypeStruct((_NC, N_NODES, D), jnp.float32),
                  jax.ShapeDtypeStruct((N_NODES,), jnp.float32),
                  jax.ShapeDtypeStruct((N_NODES,), jnp.float32)],
        mesh=mesh,
        scratch_types=[
            pltpu.VMEM_SHARED((N_NODES, D), jnp.float32),
            pltpu.VMEM_SHARED((_NS * N_NODES,), jnp.float32),
            pltpu.VMEM((_CH,), jnp.int32),
            pltpu.VMEM((_CH,), jnp.int32),
            pltpu.VMEM((_CH, D), jnp.float32),
            pltpu.VMEM((_HPAD,), jnp.float32),
            pltpu.VMEM((2 * _RPT,), jnp.float32),
            pltpu.SemaphoreType.DMA,
        ],
    )
    return fn(X, src, dst, z128)

_RB1 = 1000
_RB2 = 1000


def _sage_dense_body(x_ref, aggp_ref, degp_ref, ws_ref, wn_ref, b_ref, o_ref):
    agg = aggp_ref[0] + aggp_ref[1]
    deg = degp_ref[:, 0] + degp_ref[:, 1]
    inv = 1.0 / jnp.clip(deg, 1.0, None)
    hn = agg * inv[:, None]
    h = (jnp.dot(x_ref[...], ws_ref[...], preferred_element_type=jnp.float32)
         + jnp.dot(hn, wn_ref[...], preferred_element_type=jnp.float32)
         + b_ref[...])
    o_ref[...] = jnp.where(h >= 0.0, h, 0.01 * h)


def _sage_dense(X, agg_parts, deg_parts, W_self, W_neigh, b_sage):
    grid = (N_NODES // _RB1,)
    return pl.pallas_call(
        _sage_dense_body,
        grid=grid,
        in_specs=[
            pl.BlockSpec((_RB1, D), lambda i: (i, 0)),
            pl.BlockSpec((2, _RB1, D), lambda i: (0, i, 0)),
            pl.BlockSpec((_RB1, 2), lambda i: (i, 0)),
            pl.BlockSpec((D, D), lambda i: (0, 0)),
            pl.BlockSpec((D, D), lambda i: (0, 0)),
            pl.BlockSpec((1, D), lambda i: (0, 0)),
        ],
        out_specs=pl.BlockSpec((_RB1, D), lambda i: (i, 0)),
        out_shape=jax.ShapeDtypeStruct((N_NODES, D), jnp.float32),
    )(X, agg_parts, deg_parts, W_self, W_neigh, b_sage.reshape(1, D))


def _mlp_body(xx_ref, w1_ref, b1_ref, w2_ref, b2_ref, o_ref):
    l1 = (jnp.dot(xx_ref[...], w1_ref[...], preferred_element_type=jnp.float32)
          + b1_ref[...])
    l1 = jnp.where(l1 >= 0.0, l1, 0.01 * l1)
    o_ref[...] = jnp.tanh(
        jnp.dot(l1, w2_ref[...], preferred_element_type=jnp.float32) + b2_ref[...])


def _mlp(xx, W1, b1, W2, b2):
    grid = (N_NET // _RB2,)
    return pl.pallas_call(
        _mlp_body,
        grid=grid,
        in_specs=[
            pl.BlockSpec((_RB2, D), lambda i: (i, 0)),
            pl.BlockSpec((D, H1), lambda i: (0, 0)),
            pl.BlockSpec((1, H1), lambda i: (0, 0)),
            pl.BlockSpec((H1, 1), lambda i: (0, 0)),
            pl.BlockSpec((1, 1), lambda i: (0, 0)),
        ],
        out_specs=pl.BlockSpec((_RB2, 1), lambda i: (i, 0)),
        out_shape=jax.ShapeDtypeStruct((N_NET, 1), jnp.float32),
    )(xx, W1, b1.reshape(1, H1), W2, b2.reshape(1, 1))


def kernel(X, W_self, W_neigh, b_sage, W1, b1, W2, b2, edge_index, conn_src, conn_dst):
    src = edge_index[0]
    dst = edge_index[1]
    agg_parts, deg0, deg1 = _seg_sum_sc(X, src, dst)
    deg_parts = jnp.stack([deg0, deg1], axis=1)
    h = _sage_dense(X, agg_parts, deg_parts, W_self, W_neigh, b_sage)
    m = jnp.take(h, conn_src, axis=0)
    xx = jax.ops.segment_max(m, conn_dst, num_segments=N_NET)
    xx = jnp.where(jnp.isneginf(xx), 0.0, xx)
    return _mlp(xx, W1, b1, W2, b2)
